# baseline (speedup 1.0000x reference)
import functools

import jax
import jax.numpy as jnp
from jax import lax
from jax.experimental import pallas as pl
from jax.experimental.pallas import tpu as pltpu
from jax.experimental.pallas import tpu_sc as plsc

MEM_ROWS = 100000
BATCH_ROWS = 16384
DIM = 128
NUM_WORKERS = 32
EMB_PER_W = BATCH_ROWS // NUM_WORKERS
TAIL_ROWS = MEM_ROWS - BATCH_ROWS
TAIL_PER_W = (TAIL_ROWS // NUM_WORKERS) // 8 * 8
REM_BASE = BATCH_ROWS + NUM_WORKERS * TAIL_PER_W
REM_ROWS = MEM_ROWS - REM_BASE
REM_CHUNK = 8
REM_WORKERS = REM_ROWS // REM_CHUNK

_mesh = plsc.VectorSubcoreMesh(core_axis_name="c", subcore_axis_name="s")


@functools.partial(
    pl.kernel,
    mesh=_mesh,
    out_type=(
        jax.ShapeDtypeStruct((MEM_ROWS, DIM), jnp.float32),
        jax.ShapeDtypeStruct((MEM_ROWS, DIM), jnp.float32),
    ),
    scratch_types=[
        pltpu.SemaphoreType.DMA,
        pltpu.SemaphoreType.DMA,
        pltpu.SemaphoreType.DMA,
        pltpu.SemaphoreType.DMA,
        pltpu.SemaphoreType.DMA,
        pltpu.SemaphoreType.DMA,
    ],
)
def _bank_update(emb_ta, emb_tv, bank_ta, bank_tv, out_ta, out_tv,
                 sem0, sem1, sem2, sem3, sem4, sem5):
    wid = lax.axis_index("s") * 2 + lax.axis_index("c")
    e0 = wid * EMB_PER_W
    t0 = BATCH_ROWS + wid * TAIL_PER_W
    r0 = REM_BASE + wid * REM_CHUNK
    c0 = pltpu.make_async_copy(
        emb_ta.at[pl.ds(e0, EMB_PER_W)], out_ta.at[pl.ds(e0, EMB_PER_W)], sem0)
    c1 = pltpu.make_async_copy(
        emb_tv.at[pl.ds(e0, EMB_PER_W)], out_tv.at[pl.ds(e0, EMB_PER_W)], sem1)
    c2 = pltpu.make_async_copy(
        bank_ta.at[pl.ds(t0, TAIL_PER_W)], out_ta.at[pl.ds(t0, TAIL_PER_W)], sem2)
    c3 = pltpu.make_async_copy(
        bank_tv.at[pl.ds(t0, TAIL_PER_W)], out_tv.at[pl.ds(t0, TAIL_PER_W)], sem3)
    c0.start()
    c1.start()
    c2.start()
    c3.start()

    @pl.when(wid < REM_WORKERS)
    def _rem():
        c4 = pltpu.make_async_copy(
            bank_ta.at[pl.ds(r0, REM_CHUNK)], out_ta.at[pl.ds(r0, REM_CHUNK)],
            sem4)
        c5 = pltpu.make_async_copy(
            bank_tv.at[pl.ds(r0, REM_CHUNK)], out_tv.at[pl.ds(r0, REM_CHUNK)],
            sem5)
        c4.start()
        c5.start()
        c4.wait()
        c5.wait()

    c0.wait()
    c1.wait()
    c2.wait()
    c3.wait()


def kernel(embeddings_ta_neg, embeddings_tv_neg, memory_bank_ta, memory_bank_tv):
    return _bank_update(embeddings_ta_neg, embeddings_tv_neg,
                        memory_bank_ta, memory_bank_tv)
